# Initial kernel scaffold; baseline (speedup 1.0000x reference)
import jax
import jax.numpy as jnp
from jax.experimental import pallas as pl


def kernel(input_ids, table, offsets):
    raise NotImplementedError("write your pallas kernel here")


# speedup vs baseline: 24.5108x; 24.5108x over previous
import functools

import jax
import jax.numpy as jnp
from jax import lax
from jax.experimental import pallas as pl
from jax.experimental.pallas import tpu as pltpu
from jax.experimental.pallas import tpu_sc as plsc

NUM_CORES = 2
NUM_SUBCORES = 16
LANES = 16
NUM_WORKERS = NUM_CORES * NUM_SUBCORES


@functools.cache
def _build(n_total: int, d: int, chunk: int):
    n_per_w = n_total // NUM_WORKERS
    n_chunks = n_per_w // chunk
    mesh = plsc.VectorSubcoreMesh(
        core_axis_name="c", subcore_axis_name="s")

    def body(ids_hbm, off_hbm, table_hbm, out_hbm, idx_v, rows_v, off_v, sem):
        wid = lax.axis_index("s") * NUM_CORES + lax.axis_index("c")
        base = wid * n_per_w
        pltpu.sync_copy(off_hbm, off_v)
        off16 = off_v[...]

        def chunk_body(ci, _):
            cb = base + ci * chunk
            pltpu.sync_copy(ids_hbm.at[pl.ds(cb, chunk)], idx_v)

            def add_body(j, _):
                sl = pl.ds(j * LANES, LANES)
                idx_v[sl] = idx_v[sl] + off16
                return ()

            lax.fori_loop(0, chunk // LANES, add_body, ())
            pltpu.async_copy(table_hbm.at[idx_v], rows_v, sem).wait()
            pltpu.sync_copy(rows_v, out_hbm.at[pl.ds(cb, chunk)])
            return ()

        lax.fori_loop(0, n_chunks, chunk_body, ())

    return pl.kernel(
        body,
        out_type=jax.ShapeDtypeStruct((n_total, d), jnp.float32),
        mesh=mesh,
        scratch_types=[
            pltpu.VMEM((chunk,), jnp.int32),
            pltpu.VMEM((chunk, d), jnp.float32),
            pltpu.VMEM((LANES,), jnp.int32),
            pltpu.SemaphoreType.DMA,
        ],
        compiler_params=pltpu.CompilerParams(use_tc_tiling_on_sc=False),
    )


def kernel(input_ids, table, offsets):
    b, l, h = input_ids.shape
    d = table.shape[1]
    n_total = b * l * h
    ids_flat = input_ids.reshape(n_total).astype(jnp.int32)
    off16 = jnp.tile(offsets.astype(jnp.int32), LANES // h)
    out = _build(n_total, d, 1024)(ids_flat, off16, table)
    return out.reshape(b, l, h, d)


# speedup vs baseline: 25.4410x; 1.0380x over previous
import functools

import jax
import jax.numpy as jnp
from jax import lax
from jax.experimental import pallas as pl
from jax.experimental.pallas import tpu as pltpu
from jax.experimental.pallas import tpu_sc as plsc

NUM_CORES = 2
NUM_SUBCORES = 16
LANES = 16
NUM_WORKERS = NUM_CORES * NUM_SUBCORES


@functools.cache
def _build(n_total: int, d: int, chunk: int):
    n_per_w = n_total // NUM_WORKERS
    n_chunks = n_per_w // chunk
    assert n_chunks % 2 == 0 and n_chunks >= 4
    mesh = plsc.VectorSubcoreMesh(
        core_axis_name="c", subcore_axis_name="s")

    def body(ids_hbm, off_hbm, table_hbm, out_hbm,
             idx0, idx1, rows0, rows1, off_v,
             isem0, isem1, gsem, wsem0, wsem1):
        idx = (idx0, idx1)
        rows = (rows0, rows1)
        isem = (isem0, isem1)
        wsem = (wsem0, wsem1)
        wid = lax.axis_index("s") * NUM_CORES + lax.axis_index("c")
        base = wid * n_per_w
        pltpu.sync_copy(off_hbm, off_v)
        off16 = off_v[...]

        def ids_slice(ci):
            return ids_hbm.at[pl.ds(base + ci * chunk, chunk)]

        def out_slice(ci):
            return out_hbm.at[pl.ds(base + ci * chunk, chunk)]

        def add_offsets(b):
            for j in range(chunk // LANES):
                sl = pl.ds(j * LANES, LANES)
                idx[b][sl] = idx[b][sl] + off16

        pltpu.async_copy(ids_slice(0), idx[0], isem[0]).wait()
        add_offsets(0)

        def pair_body(cp, _):
            for b in (0, 1):
                ci = cp * 2 + b
                pb = 1 - b

                @pl.when(ci > 0)
                def _():
                    pltpu.make_async_copy(
                        table_hbm.at[idx[pb]], rows[pb], gsem).wait()
                    pltpu.async_copy(rows[pb], out_slice(ci - 1), wsem[pb])

                @pl.when(ci >= 2)
                def _():
                    pltpu.make_async_copy(
                        rows[b], out_slice(ci - 2), wsem[b]).wait()

                pltpu.async_copy(table_hbm.at[idx[b]], rows[b], gsem)

                @pl.when(ci + 1 < n_chunks)
                def _():
                    pltpu.async_copy(
                        ids_slice(ci + 1), idx[pb], isem[pb]).wait()
                    add_offsets(pb)
            return ()

        lax.fori_loop(0, n_chunks // 2, pair_body, ())

        last = n_chunks - 1
        pltpu.make_async_copy(table_hbm.at[idx[1]], rows[1], gsem).wait()
        pltpu.async_copy(rows[1], out_slice(last), wsem[1])
        pltpu.make_async_copy(rows[0], out_slice(last - 1), wsem[0]).wait()
        pltpu.make_async_copy(rows[1], out_slice(last), wsem[1]).wait()

    return pl.kernel(
        body,
        out_type=jax.ShapeDtypeStruct((n_total, d), jnp.float32),
        mesh=mesh,
        scratch_types=[
            pltpu.VMEM((chunk,), jnp.int32),
            pltpu.VMEM((chunk,), jnp.int32),
            pltpu.VMEM((chunk, d), jnp.float32),
            pltpu.VMEM((chunk, d), jnp.float32),
            pltpu.VMEM((LANES,), jnp.int32),
            pltpu.SemaphoreType.DMA,
            pltpu.SemaphoreType.DMA,
            pltpu.SemaphoreType.DMA,
            pltpu.SemaphoreType.DMA,
            pltpu.SemaphoreType.DMA,
        ],
        compiler_params=pltpu.CompilerParams(use_tc_tiling_on_sc=False),
    )


def kernel(input_ids, table, offsets):
    b, l, h = input_ids.shape
    d = table.shape[1]
    n_total = b * l * h
    ids_flat = input_ids.reshape(n_total).astype(jnp.int32)
    off16 = jnp.tile(offsets.astype(jnp.int32), LANES // h)
    out = _build(n_total, d, 1600)(ids_flat, off16, table)
    return out.reshape(b, l, h, d)
